# baseline (speedup 1.0000x reference)
import functools

import jax
import jax.numpy as jnp
from jax import lax
from jax.experimental import pallas as pl
from jax.experimental.pallas import tpu as pltpu
from jax.experimental.pallas import tpu_sc as plsc

N = 10000
D = 128
R = 3
NB = 2
E = 106667

NW = 16
ROW = 128
NCH_D = 53
NCH_PAD_D = 56
E_PAD = NW * NCH_D * ROW
CHUNK = 128
NCH_A = NCH_D
BLK_A = NCH_PAD_D
N_PAD = 10240
RPS = N_PAD // 16


def _agg_body(x_hbm, src_hbm, dst_flat, agg_out, deg_out, idx_src, idx1, idx1b,
              rows, dbounce, agg_sh, sem, sem2, semi):
    c = lax.axis_index("c")
    s = lax.axis_index("s")
    w = c * 16 + s
    zf = jnp.zeros((16,), jnp.float32)
    onef = jnp.ones((16,), jnp.float32)
    base = s * RPS

    def fill(val):
        def body(i, carry):
            for j in range(8):
                rows[i, pl.ds(j * 16, 16)] = val
            return carry
        lax.fori_loop(0, CHUNK, body, 0)

    for r in range(R):
        fill(zf)
        for h in range(RPS // CHUNK):
            pltpu.sync_copy(rows, agg_sh.at[pl.ds(base + h * CHUNK, CHUNK)])
        plsc.subcore_barrier()

        pltpu.sync_copy(src_hbm.at[r, pl.ds(w * BLK_A, BLK_A)], idx_src)
        dbase = (r * NW + w) * BLK_A * CHUNK

        pltpu.sync_copy(dst_flat.at[pl.ds(dbase, CHUNK)], idx1)

        def chunk_pair(t, carry):
            j0 = 2 * t
            pltpu.async_copy(dst_flat.at[pl.ds(dbase + (j0 + 1) * CHUNK,
                                               CHUNK)], idx1b, semi)
            pltpu.async_copy(x_hbm.at[idx_src.at[j0]], rows, sem).wait()
            pltpu.async_copy(rows, agg_sh.at[idx1], sem2, add=True).wait()
            pltpu.make_async_copy(dst_flat.at[pl.ds(dbase, CHUNK)],
                                  idx1b, semi).wait()
            pltpu.async_copy(dst_flat.at[pl.ds(dbase + (j0 + 2) * CHUNK,
                                               CHUNK)], idx1, semi)
            pltpu.async_copy(x_hbm.at[idx_src.at[j0 + 1]], rows, sem).wait()
            pltpu.async_copy(rows, agg_sh.at[idx1b], sem2, add=True).wait()
            pltpu.make_async_copy(dst_flat.at[pl.ds(dbase, CHUNK)],
                                  idx1, semi).wait()
            return carry

        lax.fori_loop(0, (NCH_A + 1) // 2, chunk_pair, 0)
        plsc.subcore_barrier()

        for h in range(RPS // CHUNK):
            sl = pl.ds(base + h * CHUNK, CHUNK)
            pltpu.sync_copy(agg_sh.at[sl], rows)
            pltpu.sync_copy(rows, agg_out.at[c, r, sl])
        plsc.subcore_barrier()

    for r in range(R):
        fill(zf)
        for h in range(RPS // CHUNK):
            pltpu.sync_copy(rows, agg_sh.at[pl.ds(base + h * CHUNK, CHUNK)])
        plsc.subcore_barrier()

        fill(onef)
        dbase = (r * NW + w) * BLK_A * CHUNK

        pltpu.sync_copy(dst_flat.at[pl.ds(dbase, CHUNK)], idx1)

        def deg_pair(t, carry):
            j0 = 2 * t
            pltpu.async_copy(dst_flat.at[pl.ds(dbase + (j0 + 1) * CHUNK,
                                               CHUNK)], idx1b, semi)
            pltpu.async_copy(rows, agg_sh.at[idx1], sem2, add=True).wait()
            pltpu.make_async_copy(dst_flat.at[pl.ds(dbase, CHUNK)],
                                  idx1b, semi).wait()
            pltpu.async_copy(dst_flat.at[pl.ds(dbase + (j0 + 2) * CHUNK,
                                               CHUNK)], idx1, semi)
            pltpu.async_copy(rows, agg_sh.at[idx1b], sem2, add=True).wait()
            pltpu.make_async_copy(dst_flat.at[pl.ds(dbase, CHUNK)],
                                  idx1, semi).wait()
            return carry

        lax.fori_loop(0, (NCH_A + 1) // 2, deg_pair, 0)
        plsc.subcore_barrier()

        for h in range(RPS // CHUNK):
            sl = pl.ds(base + h * CHUNK, CHUNK)
            pltpu.sync_copy(agg_sh.at[sl], rows)

            def take_col(i, carry):
                dbounce[i, :] = rows[i, pl.ds(0, 16)]
                return carry

            lax.fori_loop(0, CHUNK, take_col, 0)
            pltpu.sync_copy(dbounce, deg_out.at[c, r, sl])
        plsc.subcore_barrier()


_sc_agg = functools.partial(
    pl.kernel,
    out_type=(
        jax.ShapeDtypeStruct((1, R, N_PAD, D), jnp.float32),
        jax.ShapeDtypeStruct((1, R, N_PAD, 16), jnp.float32),
    ),
    mesh=plsc.VectorSubcoreMesh(core_axis_name="c", subcore_axis_name="s", num_cores=1),
    scratch_types=[
        pltpu.VMEM((BLK_A, CHUNK), jnp.int32),
        pltpu.VMEM((CHUNK,), jnp.int32),
        pltpu.VMEM((CHUNK,), jnp.int32),
        pltpu.VMEM((CHUNK, D), jnp.float32),
        pltpu.VMEM((CHUNK, 16), jnp.float32),
        pltpu.VMEM_SHARED((N_PAD, D), jnp.float32),
        pltpu.SemaphoreType.DMA,
        pltpu.SemaphoreType.DMA,
        pltpu.SemaphoreType.DMA,
    ],
)(_agg_body)


def _tc_body(wc_ref, agg_ref, deg_ref, basis_ref, bias_ref, out_ref):
    zs = []
    for r in range(R):
        agg = agg_ref[0, r]
        deg = deg_ref[0, r, :, 0:1]
        zs.append(agg / jnp.maximum(deg, 1.0))
    acc = None
    for b in range(NB):
        y = wc_ref[0, b] * zs[0] + wc_ref[1, b] * zs[1] + wc_ref[2, b] * zs[2]
        t = jnp.dot(y, basis_ref[b], preferred_element_type=jnp.float32)
        acc = t if acc is None else acc + t
    out_ref[...] = acc + bias_ref[0]


def _pad_edges(e):
    pad = E_PAD - E
    src = jnp.concatenate([e[0], jnp.zeros((pad,), jnp.int32)])
    dst = jnp.concatenate([e[1], jnp.full((pad,), N, jnp.int32)])
    padc = (NCH_PAD_D - NCH_D) * ROW
    src = jnp.pad(src.reshape(NW, NCH_D * ROW), ((0, 0), (0, padc)))
    dst = jnp.pad(dst.reshape(NW, NCH_D * ROW), ((0, 0), (0, padc)),
                  constant_values=N)
    return (src.reshape(NW * NCH_PAD_D, ROW),
            dst.reshape(NW * NCH_PAD_D, ROW))


def kernel(x, edge_index_r0, edge_index_r1, edge_index_r2, w_comp, basis_w, bias):
    srcs, dsts = zip(*(_pad_edges(e) for e in
                       (edge_index_r0, edge_index_r1, edge_index_r2)))
    src_all = jnp.stack(srcs)
    dst_all = jnp.stack(dsts)
    src_a = src_all.reshape(R, NW * BLK_A, CHUNK)
    dst_a = dst_all.reshape(R, NW * BLK_A, CHUNK)

    dst_flat = dst_all.reshape(-1)
    agg_parts, deg_parts = _sc_agg(x, src_a, dst_flat)

    blk = 512
    out_pad = pl.pallas_call(
        _tc_body,
        grid=(N_PAD // blk,),
        in_specs=[
            pl.BlockSpec(memory_space=pltpu.MemorySpace.SMEM),
            pl.BlockSpec((1, R, blk, D), lambda i: (0, 0, i, 0)),
            pl.BlockSpec((1, R, blk, 16), lambda i: (0, 0, i, 0)),
            pl.BlockSpec((NB, D, D), lambda i: (0, 0, 0)),
            pl.BlockSpec((1, D), lambda i: (0, 0)),
        ],
        out_specs=pl.BlockSpec((blk, D), lambda i: (i, 0)),
        out_shape=jax.ShapeDtypeStruct((N_PAD, D), jnp.float32),
    )(w_comp, agg_parts, deg_parts, basis_w, bias.reshape(1, D))
    return out_pad[:N]


# speedup vs baseline: 1.1105x; 1.1105x over previous
import functools

import jax
import jax.numpy as jnp
from jax import lax
from jax.experimental import pallas as pl
from jax.experimental.pallas import tpu as pltpu
from jax.experimental.pallas import tpu_sc as plsc

N = 10000
D = 128
R = 3
NB = 2
E = 106667

NW = 16
ROW = 128
NCH_D = 53
NCH_PAD_D = 56
E_PAD = NW * NCH_D * ROW
CHUNK = 128
NCH_A = NCH_D
BLK_A = NCH_PAD_D
N_PAD = 10240
RPS = N_PAD // 16


def _agg_body(x_hbm, src_hbm, dst_flat, agg_out, deg_out, idx_src, idx1, rows,
              dbounce, agg_sh, sem, sem2):
    c = lax.axis_index("c")
    s = lax.axis_index("s")
    w = c * 16 + s
    zf = jnp.zeros((16,), jnp.float32)
    onef = jnp.ones((16,), jnp.float32)
    base = s * RPS

    def fill(val):
        def body(i, carry):
            for j in range(8):
                rows[i, pl.ds(j * 16, 16)] = val
            return carry
        lax.fori_loop(0, CHUNK, body, 0)

    for r in range(R):
        fill(zf)
        for h in range(RPS // CHUNK):
            pltpu.sync_copy(rows, agg_sh.at[pl.ds(base + h * CHUNK, CHUNK)])
        plsc.subcore_barrier()

        pltpu.sync_copy(src_hbm.at[r, pl.ds(w * BLK_A, BLK_A)], idx_src)
        dbase = (r * NW + w) * BLK_A * CHUNK

        def chunk_body(j, carry):
            pltpu.sync_copy(dst_flat.at[pl.ds(dbase + j * CHUNK, CHUNK)], idx1)
            pltpu.async_copy(x_hbm.at[idx_src.at[j]], rows, sem).wait()
            pltpu.async_copy(rows, agg_sh.at[idx1], sem2, add=True).wait()
            return carry

        lax.fori_loop(0, NCH_A, chunk_body, 0)
        plsc.subcore_barrier()

        for h in range(RPS // CHUNK):
            sl = pl.ds(base + h * CHUNK, CHUNK)
            pltpu.sync_copy(agg_sh.at[sl], rows)
            pltpu.sync_copy(rows, agg_out.at[c, r, sl])
        plsc.subcore_barrier()

    for r in range(R):
        fill(zf)
        for h in range(RPS // CHUNK):
            pltpu.sync_copy(rows, agg_sh.at[pl.ds(base + h * CHUNK, CHUNK)])
        plsc.subcore_barrier()

        fill(onef)
        dbase = (r * NW + w) * BLK_A * CHUNK

        def deg_body(j, carry):
            pltpu.sync_copy(dst_flat.at[pl.ds(dbase + j * CHUNK, CHUNK)], idx1)
            pltpu.async_copy(rows, agg_sh.at[idx1], sem2, add=True).wait()
            return carry

        lax.fori_loop(0, NCH_A, deg_body, 0)
        plsc.subcore_barrier()

        for h in range(RPS // CHUNK):
            sl = pl.ds(base + h * CHUNK, CHUNK)
            pltpu.sync_copy(agg_sh.at[sl], rows)

            def take_col(i, carry):
                dbounce[i, :] = rows[i, pl.ds(0, 16)]
                return carry

            lax.fori_loop(0, CHUNK, take_col, 0)
            pltpu.sync_copy(dbounce, deg_out.at[c, r, sl])
        plsc.subcore_barrier()


_sc_agg = functools.partial(
    pl.kernel,
    out_type=(
        jax.ShapeDtypeStruct((1, R, N_PAD, D), jnp.float32),
        jax.ShapeDtypeStruct((1, R, N_PAD, 16), jnp.float32),
    ),
    mesh=plsc.VectorSubcoreMesh(core_axis_name="c", subcore_axis_name="s", num_cores=1),
    scratch_types=[
        pltpu.VMEM((BLK_A, CHUNK), jnp.int32),
        pltpu.VMEM((CHUNK,), jnp.int32),
        pltpu.VMEM((CHUNK, D), jnp.float32),
        pltpu.VMEM((CHUNK, 16), jnp.float32),
        pltpu.VMEM_SHARED((N_PAD, D), jnp.float32),
        pltpu.SemaphoreType.DMA,
        pltpu.SemaphoreType.DMA,
    ],
)(_agg_body)


def _tc_body(wc_ref, agg_ref, deg_ref, basis_ref, bias_ref, out_ref):
    zs = []
    for r in range(R):
        agg = agg_ref[0, r]
        deg = deg_ref[0, r, :, 0:1]
        zs.append(agg / jnp.maximum(deg, 1.0))
    acc = None
    for b in range(NB):
        y = wc_ref[0, b] * zs[0] + wc_ref[1, b] * zs[1] + wc_ref[2, b] * zs[2]
        t = jnp.dot(y, basis_ref[b], preferred_element_type=jnp.float32)
        acc = t if acc is None else acc + t
    out_ref[...] = acc + bias_ref[0]


def _pad_edges(e):
    pad = E_PAD - E
    src = jnp.concatenate([e[0], jnp.zeros((pad,), jnp.int32)])
    dst = jnp.concatenate([e[1], jnp.full((pad,), N, jnp.int32)])
    padc = (NCH_PAD_D - NCH_D) * ROW
    src = jnp.pad(src.reshape(NW, NCH_D * ROW), ((0, 0), (0, padc)))
    dst = jnp.pad(dst.reshape(NW, NCH_D * ROW), ((0, 0), (0, padc)),
                  constant_values=N)
    return (src.reshape(NW * NCH_PAD_D, ROW),
            dst.reshape(NW * NCH_PAD_D, ROW))


def kernel(x, edge_index_r0, edge_index_r1, edge_index_r2, w_comp, basis_w, bias):
    srcs, dsts = zip(*(_pad_edges(e) for e in
                       (edge_index_r0, edge_index_r1, edge_index_r2)))
    src_all = jnp.stack(srcs)
    dst_all = jnp.stack(dsts)
    src_a = src_all.reshape(R, NW * BLK_A, CHUNK)
    dst_a = dst_all.reshape(R, NW * BLK_A, CHUNK)

    dst_flat = dst_all.reshape(-1)
    agg_parts, deg_parts = _sc_agg(x, src_a, dst_flat)

    blk = 512
    out_pad = pl.pallas_call(
        _tc_body,
        grid=(N_PAD // blk,),
        in_specs=[
            pl.BlockSpec(memory_space=pltpu.MemorySpace.SMEM),
            pl.BlockSpec((1, R, blk, D), lambda i: (0, 0, i, 0)),
            pl.BlockSpec((1, R, blk, 16), lambda i: (0, 0, i, 0)),
            pl.BlockSpec((NB, D, D), lambda i: (0, 0, 0)),
            pl.BlockSpec((1, D), lambda i: (0, 0)),
        ],
        out_specs=pl.BlockSpec((blk, D), lambda i: (i, 0)),
        out_shape=jax.ShapeDtypeStruct((N_PAD, D), jnp.float32),
    )(w_comp, agg_parts, deg_parts, basis_w, bias.reshape(1, D))
    return out_pad[:N]


# speedup vs baseline: 1.2962x; 1.1672x over previous
import functools

import jax
import jax.numpy as jnp
from jax import lax
from jax.experimental import pallas as pl
from jax.experimental.pallas import tpu as pltpu
from jax.experimental.pallas import tpu_sc as plsc

N = 10000
D = 128
R = 3
NB = 2
E = 106667

NW = 16
ROW = 128
NCH_D = 53
NCH_PAD_D = 56
E_PAD = NW * NCH_D * ROW
CHUNK = 128
NCH_A = NCH_D
BLK_A = NCH_PAD_D
N_PAD = 10240
RPS = N_PAD // 16


def _agg_body(x_hbm, src_hbm, dst_hbm, agg_out, deg_out, idx_src, idx_dst, rows,
              dbounce, agg_sh, sem, sem2):
    c = lax.axis_index("c")
    s = lax.axis_index("s")
    w = c * 16 + s
    zf = jnp.zeros((16,), jnp.float32)
    onef = jnp.ones((16,), jnp.float32)
    base = s * RPS

    def fill(val):
        def body(i, carry):
            for j in range(8):
                rows[i, pl.ds(j * 16, 16)] = val
            return carry
        lax.fori_loop(0, CHUNK, body, 0)

    for r in range(R):
        fill(zf)
        for h in range(RPS // CHUNK):
            pltpu.sync_copy(rows, agg_sh.at[pl.ds(base + h * CHUNK, CHUNK)])
        plsc.subcore_barrier()

        pltpu.sync_copy(src_hbm.at[r, pl.ds(w * BLK_A, BLK_A)], idx_src)
        pltpu.sync_copy(dst_hbm.at[r, pl.ds(w * BLK_A, BLK_A)], idx_dst)

        def chunk_body(j, carry):
            pltpu.async_copy(x_hbm.at[idx_src.at[j]], rows, sem).wait()
            pltpu.async_copy(rows, agg_sh.at[idx_dst.at[j]], sem2, add=True).wait()
            return carry

        lax.fori_loop(0, NCH_A, chunk_body, 0)
        plsc.subcore_barrier()

        for h in range(RPS // CHUNK):
            sl = pl.ds(base + h * CHUNK, CHUNK)
            pltpu.sync_copy(agg_sh.at[sl], rows)
            pltpu.sync_copy(rows, agg_out.at[c, r, sl])
        plsc.subcore_barrier()

    for r in range(R):
        fill(zf)
        for h in range(RPS // CHUNK):
            pltpu.sync_copy(rows, agg_sh.at[pl.ds(base + h * CHUNK, CHUNK)])
        plsc.subcore_barrier()

        fill(onef)
        pltpu.sync_copy(dst_hbm.at[r, pl.ds(w * BLK_A, BLK_A)], idx_dst)

        def deg_body(j, carry):
            pltpu.async_copy(rows, agg_sh.at[idx_dst.at[j]], sem2, add=True).wait()
            return carry

        lax.fori_loop(0, NCH_A, deg_body, 0)
        plsc.subcore_barrier()

        for h in range(RPS // CHUNK):
            sl = pl.ds(base + h * CHUNK, CHUNK)
            pltpu.sync_copy(agg_sh.at[sl], rows)

            def take_col(i, carry):
                dbounce[i, :] = rows[i, pl.ds(0, 16)]
                return carry

            lax.fori_loop(0, CHUNK, take_col, 0)
            pltpu.sync_copy(dbounce, deg_out.at[c, r, sl])
        plsc.subcore_barrier()


_sc_agg = functools.partial(
    pl.kernel,
    out_type=(
        jax.ShapeDtypeStruct((1, R, N_PAD, D), jnp.float32),
        jax.ShapeDtypeStruct((1, R, N_PAD, 16), jnp.float32),
    ),
    mesh=plsc.VectorSubcoreMesh(core_axis_name="c", subcore_axis_name="s", num_cores=1),
    scratch_types=[
        pltpu.VMEM((BLK_A, CHUNK), jnp.int32),
        pltpu.VMEM((BLK_A, CHUNK), jnp.int32),
        pltpu.VMEM((CHUNK, D), jnp.float32),
        pltpu.VMEM((CHUNK, 16), jnp.float32),
        pltpu.VMEM_SHARED((N_PAD, D), jnp.float32),
        pltpu.SemaphoreType.DMA,
        pltpu.SemaphoreType.DMA,
    ],
)(_agg_body)


def _tc_body(wc_ref, agg_ref, deg_ref, basis_ref, bias_ref, out_ref):
    zs = []
    for r in range(R):
        agg = agg_ref[0, r]
        deg = deg_ref[0, r, :, 0:1]
        zs.append(agg / jnp.maximum(deg, 1.0))
    acc = None
    for b in range(NB):
        y = wc_ref[0, b] * zs[0] + wc_ref[1, b] * zs[1] + wc_ref[2, b] * zs[2]
        t = jnp.dot(y, basis_ref[b], preferred_element_type=jnp.float32)
        acc = t if acc is None else acc + t
    out_ref[...] = acc + bias_ref[0]


def _pad_edges(e):
    pad = E_PAD - E
    src = jnp.concatenate([e[0], jnp.zeros((pad,), jnp.int32)])
    dst = jnp.concatenate([e[1], jnp.full((pad,), N, jnp.int32)])
    padc = (NCH_PAD_D - NCH_D) * ROW
    src = jnp.pad(src.reshape(NW, NCH_D * ROW), ((0, 0), (0, padc)))
    dst = jnp.pad(dst.reshape(NW, NCH_D * ROW), ((0, 0), (0, padc)),
                  constant_values=N)
    return (src.reshape(NW * NCH_PAD_D, ROW),
            dst.reshape(NW * NCH_PAD_D, ROW))


def kernel(x, edge_index_r0, edge_index_r1, edge_index_r2, w_comp, basis_w, bias):
    srcs, dsts = zip(*(_pad_edges(e) for e in
                       (edge_index_r0, edge_index_r1, edge_index_r2)))
    src_all = jnp.stack(srcs)
    dst_all = jnp.stack(dsts)
    src_a = src_all.reshape(R, NW * BLK_A, CHUNK)
    dst_a = dst_all.reshape(R, NW * BLK_A, CHUNK)

    agg_parts, deg_parts = _sc_agg(x, src_a, dst_a)

    blk = 512
    out_pad = pl.pallas_call(
        _tc_body,
        grid=(N_PAD // blk,),
        in_specs=[
            pl.BlockSpec(memory_space=pltpu.MemorySpace.SMEM),
            pl.BlockSpec((1, R, blk, D), lambda i: (0, 0, i, 0)),
            pl.BlockSpec((1, R, blk, 16), lambda i: (0, 0, i, 0)),
            pl.BlockSpec((NB, D, D), lambda i: (0, 0, 0)),
            pl.BlockSpec((1, D), lambda i: (0, 0)),
        ],
        out_specs=pl.BlockSpec((blk, D), lambda i: (i, 0)),
        out_shape=jax.ShapeDtypeStruct((N_PAD, D), jnp.float32),
    )(w_comp, agg_parts, deg_parts, basis_w, bias.reshape(1, D))
    return out_pad[:N]
